# baseline (speedup 1.0000x reference)
import functools

import jax
import jax.numpy as jnp
from jax import lax
from jax.experimental import pallas as pl
from jax.experimental.pallas import tpu as pltpu
from jax.experimental.pallas import tpu_sc as plsc

N_NODES = 10000
D = 128
DH = 64
NC = 2
NS = 16
NW = NC * NS
CH = 128
AGG_ROWS = 10240
ZCOPIES = 5
OPT8 = (N_NODES // NS) // 8 * 8


def _sc_agg_call(kch):
    mesh = plsc.VectorSubcoreMesh(core_axis_name="c", subcore_axis_name="s")

    @functools.partial(
        pl.kernel,
        mesh=mesh,
        compiler_params=pltpu.CompilerParams(use_tc_tiling_on_sc=False),
        out_type=jax.ShapeDtypeStruct((NC, 2, N_NODES, DH), jnp.float32),
        scratch_types=[
            pltpu.VMEM((kch, CH), jnp.int32),
            pltpu.VMEM((kch, CH), jnp.int32),
            pltpu.VMEM((kch, CH), jnp.int32),
            pltpu.VMEM((CH, DH), jnp.float32),
            pltpu.VMEM((CH, DH), jnp.float32),
            pltpu.VMEM((CH, DH), jnp.float32),
            pltpu.VMEM((CH, DH), jnp.float32),
            pltpu.VMEM_SHARED((AGG_ROWS, DH), jnp.float32),
            pltpu.SemaphoreType.DMA,
            pltpu.SemaphoreType.DMA,
            pltpu.SemaphoreType.DMA,
            pltpu.SemaphoreType.DMA,
        ],
    )
    def sc_agg(src_hbm, typ_hbm, dst_hbm, x0_hbm, x1_hbm, r0_hbm, r1_hbm,
               part_hbm, src_v, typ_v, dst_v, xba, rba, xbb, rbb, agg_sh,
               semxa, semra, semxb, semrb):
        c = lax.axis_index("c")
        s = lax.axis_index("s")
        wid = c * NS + s

        pltpu.sync_copy(src_hbm.at[wid], src_v)
        pltpu.sync_copy(typ_hbm.at[wid], typ_v)
        pltpu.sync_copy(dst_hbm.at[wid], dst_v)

        zero = jnp.zeros((16,), jnp.float32)

        for p, (xh, rh) in enumerate(((x0_hbm, r0_hbm), (x1_hbm, r1_hbm))):
            def zrow(r, carry):
                for k in range(DH // 16):
                    xba[r, pl.ds(k * 16, 16)] = zero
                return carry

            lax.fori_loop(0, CH, zrow, 0)
            for j in range(ZCOPIES):
                pltpu.sync_copy(
                    xba, agg_sh.at[pl.ds(s * (ZCOPIES * CH) + j * CH, CH)])
            plsc.subcore_barrier()

            def start(ch, xb, rb, semx, semr):
                pltpu.async_copy(xh.at[src_v.at[ch]], xb, semx)
                pltpu.async_copy(rh.at[typ_v.at[ch]], rb, semr)

            def finish(ch, xb, rb, semx, semr):
                pltpu.make_async_copy(xh.at[src_v.at[ch]], xb, semx).wait()
                pltpu.make_async_copy(rh.at[typ_v.at[ch]], rb, semr).wait()

                def mul(r, inner):
                    for k in range(DH // 16):
                        sl = pl.ds(k * 16, 16)
                        xb[r, sl] = xb[r, sl] * rb[r, sl]
                    return inner

                lax.fori_loop(0, CH, mul, 0)
                pltpu.sync_copy(xb, agg_sh.at[dst_v.at[ch]], add=True)

            start(0, xba, rba, semxa, semra)

            def pair(i, carry):
                ch0 = i * 2
                start(ch0 + 1, xbb, rbb, semxb, semrb)
                finish(ch0, xba, rba, semxa, semra)

                @pl.when(ch0 + 2 < kch)
                def _():
                    start(ch0 + 2, xba, rba, semxa, semra)

                finish(ch0 + 1, xbb, rbb, semxb, semrb)
                return carry

            lax.fori_loop(0, kch // 2, pair, 0)
            plsc.subcore_barrier()

            pltpu.sync_copy(agg_sh.at[pl.ds(s * OPT8, OPT8)],
                            part_hbm.at[c].at[p].at[pl.ds(s * OPT8, OPT8)])

            @pl.when(s == 0)
            def _():
                base = NS * OPT8
                pltpu.sync_copy(agg_sh.at[pl.ds(base, N_NODES - base)],
                                part_hbm.at[c].at[p].at[pl.ds(base, N_NODES - base)])

            plsc.subcore_barrier()

    return sc_agg


def _tc_body(p_ref, x_ref, w_ref, b_ref, o_ref):
    agg = jnp.concatenate(
        [p_ref[0, 0] + p_ref[1, 0], p_ref[0, 1] + p_ref[1, 1]], axis=1)
    y = jnp.dot(agg, w_ref[...], preferred_element_type=jnp.float32,
                precision=lax.Precision.HIGHEST)
    o_ref[...] = jnp.maximum(y + b_ref[...] + x_ref[...], 0.0)


def _tc_update(parts, x, w, b2d):
    blk = 1000
    grid = (N_NODES // blk,)
    return pl.pallas_call(
        _tc_body,
        grid=grid,
        in_specs=[
            pl.BlockSpec((NC, 2, blk, DH), lambda i: (0, 0, i, 0)),
            pl.BlockSpec((blk, D), lambda i: (i, 0)),
            pl.BlockSpec((D, D), lambda i: (0, 0)),
            pl.BlockSpec((1, D), lambda i: (0, 0)),
        ],
        out_specs=pl.BlockSpec((blk, D), lambda i: (i, 0)),
        out_shape=jax.ShapeDtypeStruct((N_NODES, D), jnp.float32),
    )(parts, x, w, b2d)


def kernel(x, edge_index, edge_type, rel_emb, W, b):
    n_edges = edge_index.shape[1]
    kch = -(-n_edges // (NW * CH))
    kch += kch % 2
    e_pad = NW * kch * CH
    pad = e_pad - n_edges

    src = edge_index[0].astype(jnp.int32)
    dst = edge_index[1].astype(jnp.int32)
    typ = edge_type.astype(jnp.int32)
    if pad:
        src = jnp.concatenate([src, jnp.zeros((pad,), jnp.int32)])
        typ = jnp.concatenate([typ, jnp.zeros((pad,), jnp.int32)])
        dst = jnp.concatenate([dst, jnp.full((pad,), N_NODES, jnp.int32)])
    src = src.reshape(NW, kch, CH)
    typ = typ.reshape(NW, kch, CH)
    dst = dst.reshape(NW, kch, CH)

    x0 = x[:, :DH]
    x1 = x[:, DH:]
    r0 = rel_emb[:, :DH]
    r1 = rel_emb[:, DH:]

    parts = _sc_agg_call(kch)(src, typ, dst, x0, x1, r0, r1)
    return _tc_update(parts, x, W, b.reshape(1, D))


# speedup vs baseline: 1.5325x; 1.5325x over previous
import functools

import jax
import jax.numpy as jnp
from jax import lax
from jax.experimental import pallas as pl
from jax.experimental.pallas import tpu as pltpu
from jax.experimental.pallas import tpu_sc as plsc

N_NODES = 10000
D = 128
DH = 64
NC = 2
NS = 16
NW = NC * NS
CH = 128
NBUF = 4
N_REL = 51
AGG_ROWS = 10240
ZCOPIES = 5
OPT8 = (N_NODES // NS) // 8 * 8


def _sc_agg_call(kch):
    mesh = plsc.VectorSubcoreMesh(core_axis_name="c", subcore_axis_name="s")

    @functools.partial(
        pl.kernel,
        mesh=mesh,
        compiler_params=pltpu.CompilerParams(use_tc_tiling_on_sc=False),
        out_type=jax.ShapeDtypeStruct((NC, 2, N_NODES, DH), jnp.float32),
        scratch_types=[
            pltpu.VMEM((kch, CH), jnp.int32),
            pltpu.VMEM((kch, CH), jnp.int32),
            pltpu.VMEM((kch, CH), jnp.int32),
            pltpu.VMEM((N_REL, DH), jnp.float32),
            [pltpu.VMEM((CH, DH), jnp.float32) for _ in range(NBUF)],
            pltpu.VMEM_SHARED((AGG_ROWS, DH), jnp.float32),
            [pltpu.SemaphoreType.DMA for _ in range(NBUF)],
        ],
    )
    def sc_agg(src_hbm, typ_hbm, dst_hbm, x0_hbm, x1_hbm, r0_hbm, r1_hbm,
               part_hbm, src_v, typ_v, dst_v, rel_v, xbufs, agg_sh, sems):
        c = lax.axis_index("c")
        s = lax.axis_index("s")
        wid = c * NS + s

        pltpu.sync_copy(src_hbm.at[wid], src_v)
        pltpu.sync_copy(typ_hbm.at[wid], typ_v)
        pltpu.sync_copy(dst_hbm.at[wid], dst_v)

        zero = jnp.zeros((16,), jnp.float32)

        for p, (xh, rh) in enumerate(((x0_hbm, r0_hbm), (x1_hbm, r1_hbm))):
            pltpu.sync_copy(rh, rel_v)

            def zrow(r, carry):
                for k in range(DH // 16):
                    xbufs[0][r, pl.ds(k * 16, 16)] = zero
                return carry

            lax.fori_loop(0, CH, zrow, 0)
            for j in range(ZCOPIES):
                pltpu.sync_copy(
                    xbufs[0], agg_sh.at[pl.ds(s * (ZCOPIES * CH) + j * CH, CH)])
            plsc.subcore_barrier()

            def start(ch, b):
                pltpu.async_copy(xh.at[src_v.at[ch]], xbufs[b], sems[b])

            def finish(ch, b):
                pltpu.make_async_copy(
                    xh.at[src_v.at[ch]], xbufs[b], sems[b]).wait()
                xb = xbufs[b]

                def mul(g, inner):
                    tv = typ_v[ch, pl.ds(g * 16, 16)]
                    for j in range(16):
                        t = tv[j]
                        r = g * 16 + j
                        for k in range(DH // 16):
                            sl = pl.ds(k * 16, 16)
                            xb[r, sl] = xb[r, sl] * rel_v[t, sl]
                    return inner

                lax.fori_loop(0, CH // 16, mul, 0)
                pltpu.sync_copy(xb, agg_sh.at[dst_v.at[ch]], add=True)

            for b in range(NBUF - 1):
                start(b, b)

            def group(i, carry):
                for b in range(NBUF):
                    ch = i * NBUF + b

                    @pl.when(ch + NBUF - 1 < kch)
                    def _():
                        start(ch + NBUF - 1, (b + NBUF - 1) % NBUF)

                    finish(ch, b)
                return carry

            lax.fori_loop(0, kch // NBUF, group, 0)
            plsc.subcore_barrier()

            pltpu.sync_copy(agg_sh.at[pl.ds(s * OPT8, OPT8)],
                            part_hbm.at[c].at[p].at[pl.ds(s * OPT8, OPT8)])

            @pl.when(s == 0)
            def _():
                base = NS * OPT8
                pltpu.sync_copy(agg_sh.at[pl.ds(base, N_NODES - base)],
                                part_hbm.at[c].at[p].at[pl.ds(base, N_NODES - base)])

            plsc.subcore_barrier()

    return sc_agg


def _tc_body(p_ref, x_ref, w_ref, b_ref, o_ref):
    agg = jnp.concatenate(
        [p_ref[0, 0] + p_ref[1, 0], p_ref[0, 1] + p_ref[1, 1]], axis=1)
    y = jnp.dot(agg, w_ref[...], preferred_element_type=jnp.float32,
                precision=lax.Precision.HIGHEST)
    o_ref[...] = jnp.maximum(y + b_ref[...] + x_ref[...], 0.0)


def _tc_update(parts, x, w, b2d):
    blk = 1000
    grid = (N_NODES // blk,)
    return pl.pallas_call(
        _tc_body,
        grid=grid,
        in_specs=[
            pl.BlockSpec((NC, 2, blk, DH), lambda i: (0, 0, i, 0)),
            pl.BlockSpec((blk, D), lambda i: (i, 0)),
            pl.BlockSpec((D, D), lambda i: (0, 0)),
            pl.BlockSpec((1, D), lambda i: (0, 0)),
        ],
        out_specs=pl.BlockSpec((blk, D), lambda i: (i, 0)),
        out_shape=jax.ShapeDtypeStruct((N_NODES, D), jnp.float32),
    )(parts, x, w, b2d)


def kernel(x, edge_index, edge_type, rel_emb, W, b):
    n_edges = edge_index.shape[1]
    kch = -(-n_edges // (NW * CH))
    kch = -(-kch // NBUF) * NBUF
    e_pad = NW * kch * CH
    pad = e_pad - n_edges

    src = edge_index[0].astype(jnp.int32)
    dst = edge_index[1].astype(jnp.int32)
    typ = edge_type.astype(jnp.int32)
    if pad:
        src = jnp.concatenate([src, jnp.zeros((pad,), jnp.int32)])
        typ = jnp.concatenate([typ, jnp.zeros((pad,), jnp.int32)])
        dst = jnp.concatenate([dst, jnp.full((pad,), N_NODES, jnp.int32)])
    src = src.reshape(NW, kch, CH)
    typ = typ.reshape(NW, kch, CH)
    dst = dst.reshape(NW, kch, CH)

    x0 = x[:, :DH]
    x1 = x[:, DH:]
    r0 = rel_emb[:, :DH]
    r1 = rel_emb[:, DH:]

    parts = _sc_agg_call(kch)(src, typ, dst, x0, x1, r0, r1)
    return _tc_update(parts, x, W, b.reshape(1, D))


# speedup vs baseline: 1.5378x; 1.0034x over previous
import functools

import jax
import jax.numpy as jnp
import numpy as np
from jax import lax
from jax.experimental import pallas as pl
from jax.experimental.pallas import tpu as pltpu
from jax.experimental.pallas import tpu_sc as plsc

N_NODES = 10000
D = 128
DH = 64
DP = DH // 2
NC = 2
NS = 16
NW = NC * NS
CH = 128
NBUF = 8
N_REL = 51
AGG_ROWS = 10240
ZCOPIES = 5
OPT8 = (N_NODES // NS) // 8 * 8

_QHALF = np.concatenate([np.arange(0, 16), np.arange(32, 48),
                         np.arange(16, 32), np.arange(48, 64)])


def _sc_agg_call(kch):
    mesh = plsc.VectorSubcoreMesh(core_axis_name="c", subcore_axis_name="s")

    @functools.partial(
        pl.kernel,
        mesh=mesh,
        compiler_params=pltpu.CompilerParams(use_tc_tiling_on_sc=False),
        out_type=jax.ShapeDtypeStruct((NC, 2, N_NODES, DH), jnp.float32),
        scratch_types=[
            pltpu.VMEM((kch, CH), jnp.int32),
            pltpu.VMEM((kch, CH), jnp.int32),
            pltpu.VMEM((kch, CH), jnp.int32),
            pltpu.VMEM((N_REL, DH), jnp.float32),
            pltpu.VMEM((CH, DH), jnp.float32),
            [pltpu.VMEM((CH, DP), jnp.int32) for _ in range(NBUF)],
            pltpu.VMEM_SHARED((AGG_ROWS, DH), jnp.float32),
            [pltpu.SemaphoreType.DMA for _ in range(NBUF)],
        ],
    )
    def sc_agg(src_hbm, typ_hbm, dst_hbm, x0_hbm, x1_hbm, r0_hbm, r1_hbm,
               part_hbm, src_v, typ_v, dst_v, rel_v, msg_v, xbufs, agg_sh,
               sems):
        c = lax.axis_index("c")
        s = lax.axis_index("s")
        wid = c * NS + s

        pltpu.sync_copy(src_hbm.at[wid], src_v)
        pltpu.sync_copy(typ_hbm.at[wid], typ_v)
        pltpu.sync_copy(dst_hbm.at[wid], dst_v)

        zero = jnp.zeros((16,), jnp.float32)
        himask = jnp.full((16,), -65536, jnp.int32)

        for p, (xh, rh) in enumerate(((x0_hbm, r0_hbm), (x1_hbm, r1_hbm))):
            pltpu.sync_copy(rh, rel_v)

            def zrow(r, carry):
                for k in range(DH // 16):
                    msg_v[r, pl.ds(k * 16, 16)] = zero
                return carry

            lax.fori_loop(0, CH, zrow, 0)
            for j in range(ZCOPIES):
                pltpu.sync_copy(
                    msg_v, agg_sh.at[pl.ds(s * (ZCOPIES * CH) + j * CH, CH)])
            plsc.subcore_barrier()

            def start(ch, b):
                pltpu.async_copy(xh.at[src_v.at[ch]], xbufs[b], sems[b])

            def finish(ch, b):
                pltpu.make_async_copy(
                    xh.at[src_v.at[ch]], xbufs[b], sems[b]).wait()
                xb = xbufs[b]

                def mul(g, inner):
                    tv = typ_v[ch, pl.ds(g * 16, 16)]
                    for j in range(16):
                        t = tv[j]
                        r = g * 16 + j
                        for k in range(DP // 16):
                            v = xb[r, pl.ds(k * 16, 16)]
                            lo = lax.bitcast_convert_type(
                                v << 16, jnp.float32)
                            hi = lax.bitcast_convert_type(
                                v & himask, jnp.float32)
                            cl = pl.ds(k * 32, 16)
                            chh = pl.ds(k * 32 + 16, 16)
                            msg_v[r, cl] = lo * rel_v[t, cl]
                            msg_v[r, chh] = hi * rel_v[t, chh]
                    return inner

                lax.fori_loop(0, CH // 16, mul, 0)
                pltpu.sync_copy(msg_v, agg_sh.at[dst_v.at[ch]], add=True)

            for b in range(NBUF - 1):
                start(b, b)

            def group(i, carry):
                for b in range(NBUF):
                    ch = i * NBUF + b

                    @pl.when(ch + NBUF - 1 < kch)
                    def _():
                        start(ch + NBUF - 1, (b + NBUF - 1) % NBUF)

                    finish(ch, b)
                return carry

            lax.fori_loop(0, kch // NBUF, group, 0)
            plsc.subcore_barrier()

            pltpu.sync_copy(agg_sh.at[pl.ds(s * OPT8, OPT8)],
                            part_hbm.at[c].at[p].at[pl.ds(s * OPT8, OPT8)])

            @pl.when(s == 0)
            def _():
                base = NS * OPT8
                pltpu.sync_copy(agg_sh.at[pl.ds(base, N_NODES - base)],
                                part_hbm.at[c].at[p].at[pl.ds(base, N_NODES - base)])

            plsc.subcore_barrier()

    return sc_agg


def _tc_body(p_ref, x_ref, w_ref, b_ref, o_ref):
    agg = jnp.concatenate(
        [p_ref[0, 0] + p_ref[1, 0], p_ref[0, 1] + p_ref[1, 1]], axis=1)
    y = jnp.dot(agg, w_ref[...], preferred_element_type=jnp.float32,
                precision=lax.Precision.HIGHEST)
    o_ref[...] = jnp.maximum(y + b_ref[...] + x_ref[...], 0.0)


def _tc_update(parts, x, wp, b2d):
    blk = 1000
    grid = (N_NODES // blk,)
    return pl.pallas_call(
        _tc_body,
        grid=grid,
        in_specs=[
            pl.BlockSpec((NC, 2, blk, DH), lambda i: (0, 0, i, 0)),
            pl.BlockSpec((blk, D), lambda i: (i, 0)),
            pl.BlockSpec((D, D), lambda i: (0, 0)),
            pl.BlockSpec((1, D), lambda i: (0, 0)),
        ],
        out_specs=pl.BlockSpec((blk, D), lambda i: (i, 0)),
        out_shape=jax.ShapeDtypeStruct((N_NODES, D), jnp.float32),
    )(parts, x, wp, b2d)


def _pack_bf16_pairs(a, bvals):
    abits = lax.shift_right_logical(
        lax.bitcast_convert_type(a.astype(jnp.bfloat16).astype(jnp.float32),
                                 jnp.int32), 16)
    bbits = lax.bitcast_convert_type(
        bvals.astype(jnp.bfloat16).astype(jnp.float32), jnp.int32)
    return abits | (bbits & jnp.int32(-65536))


def kernel(x, edge_index, edge_type, rel_emb, W, b):
    n_edges = edge_index.shape[1]
    kch = -(-n_edges // (NW * CH))
    kch = -(-kch // NBUF) * NBUF
    e_pad = NW * kch * CH
    pad = e_pad - n_edges

    src = edge_index[0].astype(jnp.int32)
    dst = edge_index[1].astype(jnp.int32)
    typ = edge_type.astype(jnp.int32)
    if pad:
        src = jnp.concatenate([src, jnp.zeros((pad,), jnp.int32)])
        typ = jnp.concatenate([typ, jnp.zeros((pad,), jnp.int32)])
        dst = jnp.concatenate([dst, jnp.full((pad,), N_NODES, jnp.int32)])
    src = src.reshape(NW, kch, CH)
    typ = typ.reshape(NW, kch, CH)
    dst = dst.reshape(NW, kch, CH)

    x0 = _pack_bf16_pairs(x[:, 0:DP], x[:, DP:DH])
    x1 = _pack_bf16_pairs(x[:, DH:DH + DP], x[:, DH + DP:D])
    r0 = rel_emb[:, _QHALF]
    r1 = rel_emb[:, DH + _QHALF]
    wp = W[np.concatenate([_QHALF, DH + _QHALF]), :]

    parts = _sc_agg_call(kch)(src, typ, dst, x0, x1, r0, r1)
    return _tc_update(parts, x, wp, b.reshape(1, D))


# speedup vs baseline: 1.6982x; 1.1044x over previous
import functools

import jax
import jax.numpy as jnp
import numpy as np
from jax import lax
from jax.experimental import pallas as pl
from jax.experimental.pallas import tpu as pltpu
from jax.experimental.pallas import tpu_sc as plsc

N_NODES = 10000
D = 128
DH = 64
DP = DH // 2
NC = 2
NS = 16
NW = NC * NS
CH = 128
NBUF = 8
NBS = 2
N_REL = 51
AGG_ROWS = 10240
ZCOPIES = 5
OPT8 = (N_NODES // NS) // 8 * 8

_QHALF = np.concatenate([np.arange(0, 16), np.arange(32, 48),
                         np.arange(16, 32), np.arange(48, 64)])


def _sc_agg_call(kch):
    mesh = plsc.VectorSubcoreMesh(core_axis_name="c", subcore_axis_name="s")

    @functools.partial(
        pl.kernel,
        mesh=mesh,
        compiler_params=pltpu.CompilerParams(use_tc_tiling_on_sc=False),
        out_type=jax.ShapeDtypeStruct((NC, 2, N_NODES, DH), jnp.float32),
        scratch_types=[
            pltpu.VMEM((kch, CH), jnp.int32),
            pltpu.VMEM((kch, CH), jnp.int32),
            pltpu.VMEM((kch, CH), jnp.int32),
            pltpu.VMEM((N_REL, DH), jnp.float32),
            [pltpu.VMEM((CH, DH), jnp.float32) for _ in range(NBS)],
            [pltpu.VMEM((CH, DP), jnp.int32) for _ in range(NBUF)],
            pltpu.VMEM_SHARED((AGG_ROWS, DH), jnp.float32),
            [pltpu.SemaphoreType.DMA for _ in range(NBUF)],
            [pltpu.SemaphoreType.DMA for _ in range(NBS)],
        ],
    )
    def sc_agg(src_hbm, typ_hbm, dst_hbm, x0_hbm, x1_hbm, r0_hbm, r1_hbm,
               part_hbm, src_v, typ_v, dst_v, rel_v, msgbufs, xbufs, agg_sh,
               sems, ssems):
        c = lax.axis_index("c")
        s = lax.axis_index("s")
        wid = c * NS + s

        pltpu.sync_copy(src_hbm.at[wid], src_v)
        pltpu.sync_copy(typ_hbm.at[wid], typ_v)
        pltpu.sync_copy(dst_hbm.at[wid], dst_v)

        zero = jnp.zeros((16,), jnp.float32)
        himask = jnp.full((16,), -65536, jnp.int32)

        for p, (xh, rh) in enumerate(((x0_hbm, r0_hbm), (x1_hbm, r1_hbm))):
            pltpu.sync_copy(rh, rel_v)

            def zrow(r, carry):
                for k in range(DH // 16):
                    msgbufs[0][r, pl.ds(k * 16, 16)] = zero
                return carry

            lax.fori_loop(0, CH, zrow, 0)
            for j in range(ZCOPIES):
                pltpu.sync_copy(
                    msgbufs[0],
                    agg_sh.at[pl.ds(s * (ZCOPIES * CH) + j * CH, CH)])
            plsc.subcore_barrier()

            def start(ch, b):
                pltpu.async_copy(xh.at[src_v.at[ch]], xbufs[b], sems[b])

            def finish(ch, b):
                pltpu.make_async_copy(
                    xh.at[src_v.at[ch]], xbufs[b], sems[b]).wait()
                xb = xbufs[b]
                sb = b % NBS
                msg_v = msgbufs[sb]

                @pl.when(ch >= NBS)
                def _():
                    pltpu.make_async_copy(
                        msg_v, agg_sh.at[dst_v.at[ch]], ssems[sb]).wait()

                def mul(g, inner):
                    tv = typ_v[ch, pl.ds(g * 16, 16)]
                    for j in range(16):
                        t = tv[j]
                        r = g * 16 + j
                        for k in range(DP // 16):
                            v = xb[r, pl.ds(k * 16, 16)]
                            lo = lax.bitcast_convert_type(
                                v << 16, jnp.float32)
                            hi = lax.bitcast_convert_type(
                                v & himask, jnp.float32)
                            cl = pl.ds(k * 32, 16)
                            chh = pl.ds(k * 32 + 16, 16)
                            msg_v[r, cl] = lo * rel_v[t, cl]
                            msg_v[r, chh] = hi * rel_v[t, chh]
                    return inner

                lax.fori_loop(0, CH // 16, mul, 0)
                pltpu.async_copy(
                    msg_v, agg_sh.at[dst_v.at[ch]], ssems[sb], add=True)

            for b in range(NBUF - 1):
                start(b, b)

            def group(i, carry):
                for b in range(NBUF):
                    ch = i * NBUF + b

                    @pl.when(ch + NBUF - 1 < kch)
                    def _():
                        start(ch + NBUF - 1, (b + NBUF - 1) % NBUF)

                    finish(ch, b)
                return carry

            lax.fori_loop(0, kch // NBUF, group, 0)
            for sb in range(NBS):
                pltpu.make_async_copy(
                    msgbufs[sb], agg_sh.at[dst_v.at[kch - NBS + sb]],
                    ssems[sb]).wait()
            plsc.subcore_barrier()

            pltpu.sync_copy(agg_sh.at[pl.ds(s * OPT8, OPT8)],
                            part_hbm.at[c].at[p].at[pl.ds(s * OPT8, OPT8)])

            @pl.when(s == 0)
            def _():
                base = NS * OPT8
                pltpu.sync_copy(agg_sh.at[pl.ds(base, N_NODES - base)],
                                part_hbm.at[c].at[p].at[pl.ds(base, N_NODES - base)])

            plsc.subcore_barrier()

    return sc_agg


def _tc_body(p_ref, x_ref, w_ref, b_ref, o_ref):
    agg = jnp.concatenate(
        [p_ref[0, 0] + p_ref[1, 0], p_ref[0, 1] + p_ref[1, 1]], axis=1)
    y = jnp.dot(agg, w_ref[...], preferred_element_type=jnp.float32,
                precision=lax.Precision.HIGHEST)
    o_ref[...] = jnp.maximum(y + b_ref[...] + x_ref[...], 0.0)


def _tc_update(parts, x, wp, b2d):
    blk = 1000
    grid = (N_NODES // blk,)
    return pl.pallas_call(
        _tc_body,
        grid=grid,
        in_specs=[
            pl.BlockSpec((NC, 2, blk, DH), lambda i: (0, 0, i, 0)),
            pl.BlockSpec((blk, D), lambda i: (i, 0)),
            pl.BlockSpec((D, D), lambda i: (0, 0)),
            pl.BlockSpec((1, D), lambda i: (0, 0)),
        ],
        out_specs=pl.BlockSpec((blk, D), lambda i: (i, 0)),
        out_shape=jax.ShapeDtypeStruct((N_NODES, D), jnp.float32),
    )(parts, x, wp, b2d)


def _pack_bf16_pairs(a, bvals):
    abits = lax.shift_right_logical(
        lax.bitcast_convert_type(a.astype(jnp.bfloat16).astype(jnp.float32),
                                 jnp.int32), 16)
    bbits = lax.bitcast_convert_type(
        bvals.astype(jnp.bfloat16).astype(jnp.float32), jnp.int32)
    return abits | (bbits & jnp.int32(-65536))


def kernel(x, edge_index, edge_type, rel_emb, W, b):
    n_edges = edge_index.shape[1]
    kch = -(-n_edges // (NW * CH))
    kch = -(-kch // NBUF) * NBUF
    e_pad = NW * kch * CH
    pad = e_pad - n_edges

    src = edge_index[0].astype(jnp.int32)
    dst = edge_index[1].astype(jnp.int32)
    typ = edge_type.astype(jnp.int32)
    if pad:
        src = jnp.concatenate([src, jnp.zeros((pad,), jnp.int32)])
        typ = jnp.concatenate([typ, jnp.zeros((pad,), jnp.int32)])
        dst = jnp.concatenate([dst, jnp.full((pad,), N_NODES, jnp.int32)])
    src = src.reshape(NW, kch, CH)
    typ = typ.reshape(NW, kch, CH)
    dst = dst.reshape(NW, kch, CH)

    x0 = _pack_bf16_pairs(x[:, 0:DP], x[:, DP:DH])
    x1 = _pack_bf16_pairs(x[:, DH:DH + DP], x[:, DH + DP:D])
    r0 = rel_emb[:, _QHALF]
    r1 = rel_emb[:, DH + _QHALF]
    wp = W[np.concatenate([_QHALF, DH + _QHALF]), :]

    parts = _sc_agg_call(kch)(src, typ, dst, x0, x1, r0, r1)
    return _tc_update(parts, x, wp, b.reshape(1, D))


# speedup vs baseline: 1.7012x; 1.0017x over previous
import functools

import jax
import jax.numpy as jnp
import numpy as np
from jax import lax
from jax.experimental import pallas as pl
from jax.experimental.pallas import tpu as pltpu
from jax.experimental.pallas import tpu_sc as plsc

N_NODES = 10000
D = 128
DH = 64
DP = DH // 2
NC = 2
NS = 16
NW = NC * NS
CH = 128
NBUF = 8
NBS = 2
N_REL = 51
AGG_ROWS = 10240
ZCOPIES = 5
OPT8 = (N_NODES // NS) // 8 * 8

_QHALF = np.concatenate([np.arange(0, 16), np.arange(32, 48),
                         np.arange(16, 32), np.arange(48, 64)])


def _sc_agg_call(kch):
    mesh = plsc.VectorSubcoreMesh(core_axis_name="c", subcore_axis_name="s")

    @functools.partial(
        pl.kernel,
        mesh=mesh,
        compiler_params=pltpu.CompilerParams(use_tc_tiling_on_sc=False),
        out_type=jax.ShapeDtypeStruct((NC, 2, N_NODES, DH), jnp.float32),
        scratch_types=[
            pltpu.VMEM((kch, CH), jnp.int32),
            pltpu.VMEM((kch, CH), jnp.int32),
            pltpu.VMEM((kch, CH), jnp.int32),
            pltpu.VMEM((N_REL, DH), jnp.float32),
            [pltpu.VMEM((CH, DH), jnp.float32) for _ in range(NBS)],
            [pltpu.VMEM((CH, DP), jnp.int32) for _ in range(NBUF)],
            pltpu.VMEM_SHARED((AGG_ROWS, DH), jnp.float32),
            [pltpu.SemaphoreType.DMA for _ in range(NBUF)],
            [pltpu.SemaphoreType.DMA for _ in range(NBS)],
        ],
    )
    def sc_agg(src_hbm, typ_hbm, dst_hbm, x0_hbm, x1_hbm, r0_hbm, r1_hbm,
               part_hbm, src_v, typ_v, dst_v, rel_v, msgbufs, xbufs, agg_sh,
               sems, ssems):
        c = lax.axis_index("c")
        s = lax.axis_index("s")
        wid = c * NS + s

        cpi1 = pltpu.async_copy(src_hbm.at[wid], src_v, sems[0])
        cpi2 = pltpu.async_copy(typ_hbm.at[wid], typ_v, sems[1])
        cpi3 = pltpu.async_copy(dst_hbm.at[wid], dst_v, sems[2])
        cpi1.wait()
        cpi2.wait()
        cpi3.wait()

        zero = jnp.zeros((16,), jnp.float32)
        himask = jnp.full((16,), -65536, jnp.int32)

        for p, (xh, rh) in enumerate(((x0_hbm, r0_hbm), (x1_hbm, r1_hbm))):
            pltpu.sync_copy(rh, rel_v)

            def zrow(r, carry):
                for k in range(DH // 16):
                    msgbufs[0][r, pl.ds(k * 16, 16)] = zero
                return carry

            lax.fori_loop(0, CH, zrow, 0)
            zcps = [
                pltpu.async_copy(
                    msgbufs[0],
                    agg_sh.at[pl.ds(s * (ZCOPIES * CH) + j * CH, CH)],
                    sems[j])
                for j in range(ZCOPIES)
            ]
            for cp in zcps:
                cp.wait()
            plsc.subcore_barrier()

            def start(ch, b):
                pltpu.async_copy(xh.at[src_v.at[ch]], xbufs[b], sems[b])

            def finish(ch, b):
                pltpu.make_async_copy(
                    xh.at[src_v.at[ch]], xbufs[b], sems[b]).wait()
                xb = xbufs[b]
                sb = b % NBS
                msg_v = msgbufs[sb]

                @pl.when(ch >= NBS)
                def _():
                    pltpu.make_async_copy(
                        msg_v, agg_sh.at[dst_v.at[ch]], ssems[sb]).wait()

                def mul(g, inner):
                    tv = typ_v[ch, pl.ds(g * 16, 16)]
                    for j in range(16):
                        t = tv[j]
                        r = g * 16 + j
                        for k in range(DP // 16):
                            v = xb[r, pl.ds(k * 16, 16)]
                            lo = lax.bitcast_convert_type(
                                v << 16, jnp.float32)
                            hi = lax.bitcast_convert_type(
                                v & himask, jnp.float32)
                            cl = pl.ds(k * 32, 16)
                            chh = pl.ds(k * 32 + 16, 16)
                            msg_v[r, cl] = lo * rel_v[t, cl]
                            msg_v[r, chh] = hi * rel_v[t, chh]
                    return inner

                lax.fori_loop(0, CH // 16, mul, 0)
                pltpu.async_copy(
                    msg_v, agg_sh.at[dst_v.at[ch]], ssems[sb], add=True)

            for b in range(NBUF - 1):
                start(b, b)

            def group(i, carry):
                for b in range(NBUF):
                    ch = i * NBUF + b

                    @pl.when(ch + NBUF - 1 < kch)
                    def _():
                        start(ch + NBUF - 1, (b + NBUF - 1) % NBUF)

                    finish(ch, b)
                return carry

            lax.fori_loop(0, kch // NBUF, group, 0)
            for sb in range(NBS):
                pltpu.make_async_copy(
                    msgbufs[sb], agg_sh.at[dst_v.at[kch - NBS + sb]],
                    ssems[sb]).wait()
            plsc.subcore_barrier()

            pltpu.sync_copy(agg_sh.at[pl.ds(s * OPT8, OPT8)],
                            part_hbm.at[c].at[p].at[pl.ds(s * OPT8, OPT8)])

            @pl.when(s == 0)
            def _():
                base = NS * OPT8
                pltpu.sync_copy(agg_sh.at[pl.ds(base, N_NODES - base)],
                                part_hbm.at[c].at[p].at[pl.ds(base, N_NODES - base)])

            plsc.subcore_barrier()

    return sc_agg


def _tc_body(p_ref, x_ref, w_ref, b_ref, o_ref):
    agg = jnp.concatenate(
        [p_ref[0, 0] + p_ref[1, 0], p_ref[0, 1] + p_ref[1, 1]], axis=1)
    y = jnp.dot(agg, w_ref[...], preferred_element_type=jnp.float32,
                precision=lax.Precision.HIGHEST)
    o_ref[...] = jnp.maximum(y + b_ref[...] + x_ref[...], 0.0)


def _tc_update(parts, x, wp, b2d):
    blk = 1000
    grid = (N_NODES // blk,)
    return pl.pallas_call(
        _tc_body,
        grid=grid,
        in_specs=[
            pl.BlockSpec((NC, 2, blk, DH), lambda i: (0, 0, i, 0)),
            pl.BlockSpec((blk, D), lambda i: (i, 0)),
            pl.BlockSpec((D, D), lambda i: (0, 0)),
            pl.BlockSpec((1, D), lambda i: (0, 0)),
        ],
        out_specs=pl.BlockSpec((blk, D), lambda i: (i, 0)),
        out_shape=jax.ShapeDtypeStruct((N_NODES, D), jnp.float32),
    )(parts, x, wp, b2d)


def _pack_bf16_pairs(a, bvals):
    abits = lax.shift_right_logical(
        lax.bitcast_convert_type(a.astype(jnp.bfloat16).astype(jnp.float32),
                                 jnp.int32), 16)
    bbits = lax.bitcast_convert_type(
        bvals.astype(jnp.bfloat16).astype(jnp.float32), jnp.int32)
    return abits | (bbits & jnp.int32(-65536))


def kernel(x, edge_index, edge_type, rel_emb, W, b):
    n_edges = edge_index.shape[1]
    kch = -(-n_edges // (NW * CH))
    kch = -(-kch // NBUF) * NBUF
    e_pad = NW * kch * CH
    pad = e_pad - n_edges

    src = edge_index[0].astype(jnp.int32)
    dst = edge_index[1].astype(jnp.int32)
    typ = edge_type.astype(jnp.int32)
    if pad:
        src = jnp.concatenate([src, jnp.zeros((pad,), jnp.int32)])
        typ = jnp.concatenate([typ, jnp.zeros((pad,), jnp.int32)])
        dst = jnp.concatenate([dst, jnp.full((pad,), N_NODES, jnp.int32)])
    src = src.reshape(NW, kch, CH)
    typ = typ.reshape(NW, kch, CH)
    dst = dst.reshape(NW, kch, CH)

    x0 = _pack_bf16_pairs(x[:, 0:DP], x[:, DP:DH])
    x1 = _pack_bf16_pairs(x[:, DH:DH + DP], x[:, DH + DP:D])
    r0 = rel_emb[:, _QHALF]
    r1 = rel_emb[:, DH + _QHALF]
    wp = W[np.concatenate([_QHALF, DH + _QHALF]), :]

    parts = _sc_agg_call(kch)(src, typ, dst, x0, x1, r0, r1)
    return _tc_update(parts, x, wp, b.reshape(1, D))
